# baseline (speedup 1.0000x reference)
import functools

import jax
import jax.numpy as jnp
from jax import lax
from jax.experimental import pallas as pl
from jax.experimental.pallas import tpu as pltpu
from jax.experimental.pallas import tpu_sc as plsc

N = 100000
D_IN = 768
D_HID = 512
D_ATT = 256
K = 4
TILE = 2000
NEG = float("-inf")
POS = float("inf")
BIG = 2**30
DEFAULT = jax.lax.Precision.DEFAULT


def _scan_topk(S, gidx, largest: bool):
    cur = S
    sels = []
    for _ in range(K):
        v = jnp.max(cur) if largest else jnp.min(cur)
        sel = jnp.min(jnp.where(cur == v, gidx, BIG))
        cur = jnp.where(gidx == sel, NEG if largest else POS, cur)
        sels.append(sel)
    return sels


def _stream_kernel(h_ref, Wfc_ref, bfc_ref, Wa_ref, ba_ref, Wb_ref, bb_ref,
                   Wc_ref, bc_ref, shift_ref,
                   idx_ref, s_ref, acc_ref,
                   S_ref, svec_ref, accv_ref):
    i = pl.program_id(0)
    nsteps = pl.num_programs(0)

    @pl.when(i == 0)
    def _init():
        svec_ref[...] = jnp.zeros_like(svec_ref)
        accv_ref[...] = jnp.zeros_like(accv_ref)

    h1 = jnp.maximum(
        jnp.dot(h_ref[...], Wfc_ref[...], precision=DEFAULT,
                preferred_element_type=jnp.float32)
        + bfc_ref[...], 0.0)
    a = jnp.tanh(
        jnp.dot(h1, Wa_ref[...], precision=DEFAULT,
                preferred_element_type=jnp.float32)
        + ba_ref[...])
    b = jax.nn.sigmoid(
        jnp.dot(h1, Wb_ref[...], precision=DEFAULT,
                preferred_element_type=jnp.float32)
        + bb_ref[...])
    A2 = jax.lax.dot_general(
        Wc_ref[...], a * b, (((0,), (1,)), ((), ())),
        precision=DEFAULT,
        preferred_element_type=jnp.float32) + bc_ref[0, 0]

    S_ref[pl.ds(i, 1), :] = A2

    w = jnp.exp(A2 - shift_ref[0, 0])
    svec_ref[...] += w
    accv_ref[...] += jnp.dot(w, h1, precision=DEFAULT,
                             preferred_element_type=jnp.float32)

    @pl.when(i == nsteps - 1)
    def _epilogue():
        S = S_ref[...]
        gidx = (jax.lax.broadcasted_iota(jnp.int32, S.shape, 0) * TILE
                + jax.lax.broadcasted_iota(jnp.int32, S.shape, 1))
        sels = (_scan_topk(S, gidx, largest=True)
                + _scan_topk(S, gidx, largest=False))
        idx_ref[...] = jnp.stack(sels).reshape(1, 2 * K)
        s_ref[...] = jnp.sum(svec_ref[...]).reshape(1, 1)
        acc_ref[...] = accv_ref[...]


def _sc_gather_kernel(idx_hbm, h_hbm, out_hbm, idx_v, rows_v, sem):
    wid = lax.axis_index("s") * 2 + lax.axis_index("c")

    @pl.when(wid == 0)
    def _():
        pltpu.sync_copy(idx_hbm, idx_v)
        pltpu.async_copy(h_hbm.at[idx_v], rows_v, sem).wait()
        pltpu.sync_copy(rows_v, out_hbm)


def _final_kernel(rows_ref, Wfc_ref, bfc_ref,
                  Wcls_ref, bcls_ref, Wi_ref, bi_ref,
                  s_ref, acc_ref, out_ref):
    h1 = jnp.maximum(
        jnp.dot(rows_ref[...], Wfc_ref[...], precision=DEFAULT,
                preferred_element_type=jnp.float32)
        + bfc_ref[...], 0.0)
    inst = jnp.dot(h1, Wi_ref[...], precision=DEFAULT,
                   preferred_element_type=jnp.float32) + bi_ref[...]
    M = acc_ref[...] / s_ref[0, 0]
    bag = jnp.dot(M, Wcls_ref[...], precision=DEFAULT,
                  preferred_element_type=jnp.float32) + bcls_ref[...]
    out_ref[...] = jnp.zeros((16, 128), dtype=jnp.float32)
    out_ref[0:1, 0:2] = bag
    out_ref[1:1 + 2 * K, 0:2] = inst


@functools.partial(jax.jit, static_argnames=())
def kernel(h, W_fc, b_fc, Wa, ba, Wb, bb, Wc, bc, Wcls, bcls, Wi, bi):
    nsteps = N // TILE
    const = lambda *_: (0, 0)
    shift = (jnp.sum(jnp.abs(Wc.astype(jnp.bfloat16).astype(jnp.float32)))
             + jnp.abs(bc[0])).reshape(1, 1)

    idx, s, acc = pl.pallas_call(
        _stream_kernel,
        grid=(nsteps,),
        in_specs=[
            pl.BlockSpec((TILE, D_IN), lambda i: (i, 0)),
            pl.BlockSpec((D_IN, D_HID), const),
            pl.BlockSpec((1, D_HID), const),
            pl.BlockSpec((D_HID, D_ATT), const),
            pl.BlockSpec((1, D_ATT), const),
            pl.BlockSpec((D_HID, D_ATT), const),
            pl.BlockSpec((1, D_ATT), const),
            pl.BlockSpec((D_ATT, 1), const),
            pl.BlockSpec((1, 1), const),
            pl.BlockSpec((1, 1), const),
        ],
        out_specs=[
            pl.BlockSpec((1, 2 * K), const),
            pl.BlockSpec((1, 1), const),
            pl.BlockSpec((1, D_HID), const),
        ],
        out_shape=[
            jax.ShapeDtypeStruct((1, 2 * K), jnp.int32),
            jax.ShapeDtypeStruct((1, 1), jnp.float32),
            jax.ShapeDtypeStruct((1, D_HID), jnp.float32),
        ],
        scratch_shapes=[
            pltpu.VMEM((nsteps, TILE), jnp.float32),
            pltpu.VMEM((1, TILE), jnp.float32),
            pltpu.VMEM((1, D_HID), jnp.float32),
        ],
    )(
        h, W_fc, b_fc.reshape(1, D_HID),
        Wa, ba.reshape(1, D_ATT),
        Wb, bb.reshape(1, D_ATT),
        Wc, bc.reshape(1, 1), shift,
    )

    sc_gather = functools.partial(
        pl.kernel, mesh=plsc.VectorSubcoreMesh(core_axis_name="c",
                                               subcore_axis_name="s"),
        out_type=jax.ShapeDtypeStruct((2 * K, D_IN), jnp.float32),
        scratch_types=[
            pltpu.VMEM((2 * K,), jnp.int32),
            pltpu.VMEM((2 * K, D_IN), jnp.float32),
            pltpu.SemaphoreType.DMA,
        ],
    )(_sc_gather_kernel)
    rows = sc_gather(idx.reshape(2 * K), h)

    out = pl.pallas_call(
        _final_kernel,
        grid=(1,),
        in_specs=[
            pl.BlockSpec((2 * K, D_IN), const),
            pl.BlockSpec((D_IN, D_HID), const),
            pl.BlockSpec((1, D_HID), const),
            pl.BlockSpec((D_HID, 2), const),
            pl.BlockSpec((1, 2), const),
            pl.BlockSpec((D_HID, 2), const),
            pl.BlockSpec((1, 2), const),
            pl.BlockSpec((1, 1), const),
            pl.BlockSpec((1, D_HID), const),
        ],
        out_specs=pl.BlockSpec((16, 128), const),
        out_shape=jax.ShapeDtypeStruct((16, 128), jnp.float32),
    )(
        rows, W_fc, b_fc.reshape(1, D_HID),
        Wcls, bcls.reshape(1, 2),
        Wi, bi.reshape(1, 2),
        s, acc,
    )
    return out[0:1 + 2 * K, 0:2]
